# baseline (speedup 1.0000x reference)
import functools

import jax
import jax.numpy as jnp
from jax import lax
from jax.experimental import pallas as pl
from jax.experimental.pallas import tpu as pltpu
from jax.experimental.pallas import tpu_sc as plsc

N_USERS = 50000
N_ITEMS = 50000
D = 64
GNN_LAYERS = 2
GNN_K = 2
SIM_THR = 0.5
E_UI = 800000
E_SOC = 800000
E_NET = E_SOC + N_USERS

NSUB = 16
BLK_S = 768
BLK_G = 2048
BLK_H = 2048
CHUNK_S = NSUB * BLK_S
CHUNK_G = NSUB * BLK_G
CHUNK_H = NSUB * BLK_H
NB = 50048
STRIPE = NB // NSUB
DUMMY = 50000


def _cdiv(a, b):
    return (a + b - 1) // b


def _mesh():
    return plsc.VectorSubcoreMesh(core_axis_name="c", subcore_axis_name="s")


_CPARAMS = pltpu.CompilerParams(use_tc_tiling_on_sc=False)


@functools.lru_cache(maxsize=None)
def _segsum_kernel(b, hx):
    tsteps = b // NSUB
    assert tsteps * NSUB == b

    def body(ir, x, zeros, out, acc, iv, buf, sem):
        c = lax.axis_index("c")
        s = lax.axis_index("s")
        r0 = s * STRIPE
        pltpu.sync_copy(zeros.at[pl.ds(r0, STRIPE)], acc.at[pl.ds(r0, STRIPE)])
        plsc.subcore_barrier()

        def step(t, carry):
            blk = s * tsteps + t
            pltpu.sync_copy(ir.at[c, blk], iv)
            pltpu.async_copy(x.at[iv.at[0]], buf, sem).wait()
            pltpu.sync_copy(buf, acc.at[iv.at[1]], add=True)
            return carry

        lax.fori_loop(0, tsteps, step, 0)
        plsc.subcore_barrier()
        pltpu.sync_copy(acc.at[pl.ds(r0, STRIPE)], out.at[c, pl.ds(r0, STRIPE)])

    return pl.kernel(
        body,
        out_type=jax.ShapeDtypeStruct((2, NB, 32), jnp.float32),
        mesh=_mesh(),
        compiler_params=_CPARAMS,
        scratch_types=[
            pltpu.VMEM_SHARED((NB, 32), jnp.float32),
            pltpu.VMEM((2, BLK_S), jnp.int32),
            pltpu.VMEM((BLK_S, 32), jnp.float32),
            pltpu.SemaphoreType.DMA,
        ],
    )


@functools.lru_cache(maxsize=None)
def _histo_kernel(bh):
    tsteps = bh // NSUB
    assert tsteps * NSUB == bh

    def body(rows, ones_h, zeros, out, acc, row_v, ones_v):
        c = lax.axis_index("c")
        s = lax.axis_index("s")
        r0 = s * STRIPE
        pltpu.sync_copy(zeros.at[pl.ds(r0, STRIPE)], acc.at[pl.ds(r0, STRIPE)])
        pltpu.sync_copy(ones_h, ones_v)
        plsc.subcore_barrier()

        def step(t, carry):
            blk = s * tsteps + t
            pltpu.sync_copy(rows.at[c, blk], row_v)
            pltpu.sync_copy(ones_v, acc.at[row_v], add=True)
            return carry

        lax.fori_loop(0, tsteps, step, 0)
        plsc.subcore_barrier()
        pltpu.sync_copy(acc.at[pl.ds(r0, STRIPE)], out.at[c, pl.ds(r0, STRIPE)])

    return pl.kernel(
        body,
        out_type=jax.ShapeDtypeStruct((2, NB, 16), jnp.float32),
        mesh=_mesh(),
        compiler_params=_CPARAMS,
        scratch_types=[
            pltpu.VMEM_SHARED((NB, 16), jnp.float32),
            pltpu.VMEM((BLK_H,), jnp.int32),
            pltpu.VMEM((BLK_H, 16), jnp.float32),
        ],
    )


@functools.lru_cache(maxsize=None)
def _gather_kernel(b, hx):
    tsteps = b // NSUB
    assert tsteps * NSUB == b

    def body(gidx, x, out, idx_v, buf, sem):
        c = lax.axis_index("c")
        s = lax.axis_index("s")

        def step(t, carry):
            blk = s * tsteps + t
            pltpu.sync_copy(gidx.at[c, blk], idx_v)
            pltpu.async_copy(x.at[idx_v], buf, sem).wait()
            pltpu.sync_copy(buf, out.at[c, pl.ds(blk * BLK_G, BLK_G)])
            return carry

        lax.fori_loop(0, tsteps, step, 0)

    return pl.kernel(
        body,
        out_type=jax.ShapeDtypeStruct((2, b * BLK_G, 32), jnp.float32),
        mesh=_mesh(),
        compiler_params=_CPARAMS,
        scratch_types=[
            pltpu.VMEM((BLK_G,), jnp.int32),
            pltpu.VMEM((BLK_G, 32), jnp.float32),
            pltpu.SemaphoreType.DMA,
        ],
    )


def _pad1(x, n, val):
    e = x.shape[0]
    if n == e:
        return x
    return jnp.concatenate([x, jnp.full((n - e,), val, x.dtype)])


def _gidx(col, n, blk):
    g = _pad1(col, n, 0).astype(jnp.int32) * 2
    return jnp.stack([g, g + 1]).reshape(2, n // blk, blk)


def _sidx(col, row, n):
    g = _pad1(col, n, 0).astype(jnp.int32) * 2
    r = _pad1(row, n, DUMMY).astype(jnp.int32).reshape(n // BLK_S, BLK_S)

    def half(c):
        return jnp.stack([(g + c).reshape(n // BLK_S, BLK_S), r], axis=1)

    return jnp.stack([half(0), half(1)])


def _hidx(row, nh):
    e = row.shape[0]
    h0 = _pad1(row[: e // 2], nh, DUMMY).astype(jnp.int32)
    h1 = _pad1(row[e // 2:], nh, DUMMY).astype(jnp.int32)
    return jnp.stack([h0, h1]).reshape(2, nh // BLK_H, BLK_H)


def _lin(x, W, b):
    return x @ W.T + b


EP_UI = _cdiv(E_UI, CHUNK_S) * CHUNK_S
EP_NET = _cdiv(E_NET, CHUNK_S) * CHUNK_S
EP_G = _cdiv(2 * E_NET, CHUNK_G) * CHUNK_G
EP_H_UI = _cdiv(E_UI // 2, CHUNK_H) * CHUNK_H
EP_H_NET = _cdiv((E_NET + 1) // 2, CHUNK_H) * CHUNK_H


def kernel(user_emb, item_emb, Wuc, buc, Wic, bic, Wui, bui, Wpn, bpn, Wu, bu,
           ui_row, ui_col, soc_row, soc_col, soc_sign):
    f32 = jnp.float32
    diag = jnp.arange(N_USERS, dtype=soc_row.dtype)
    net_row = jnp.concatenate([soc_row, diag])
    net_col = jnp.concatenate([soc_col, diag])
    net_sign = jnp.concatenate([soc_sign.astype(f32), jnp.ones((N_USERS,), f32)])

    zeros32 = jnp.zeros((NB, 32), f32)
    zeros16 = jnp.zeros((NB, 16), f32)
    ones16 = jnp.ones((BLK_H, 16), f32)

    def histo(rows3):
        out = _histo_kernel(rows3.shape[1])(rows3, ones16, zeros16)
        return out[0, :N_USERS, 0] + out[1, :N_USERS, 0]

    def segsum(ir, x64):
        x2 = x64.reshape(-1, 32)
        out = _segsum_kernel(ir.shape[1], x2.shape[0])(ir, x2, zeros32)
        return jnp.concatenate([out[0, :N_USERS], out[1, :N_USERS]], axis=1)

    ir_acm = _sidx(ui_col, ui_row, EP_UI)
    ir_acmT = _sidx(ui_row, ui_col, EP_UI)
    dinv_u = (1.0 / (histo(_hidx(ui_row, EP_H_UI)) + 1e-07))[:, None]
    dinv_i = (1.0 / (histo(_hidx(ui_col, EP_H_UI)) + 1e-07))[:, None]

    ACM = lambda X: dinv_u * segsum(ir_acm, X)
    ACM_T = lambda X: dinv_i * segsum(ir_acmT, X)

    item_rep0 = ACM(item_emb)
    user_rep0 = ACM_T(user_emb)
    item_init = _lin(jnp.concatenate([item_emb, ACM_T(item_rep0)], axis=1), Wic, bic)
    user_init = _lin(jnp.concatenate([user_emb, ACM(user_rep0)], axis=1), Wuc, buc)
    item_rep = ACM(item_init)
    E0 = _lin(jnp.concatenate([user_init, item_rep], axis=1), Wui, bui)

    invn = 1.0 / jnp.maximum(jnp.sqrt(jnp.sum(item_rep * item_rep, axis=1)), 1e-08)
    irn = item_rep * invn[:, None]
    gcat = _gidx(jnp.concatenate([net_row, net_col]), EP_G, BLK_G)
    sd = _gather_kernel(EP_G // BLK_G, 2 * N_ITEMS)(gcat, irn.reshape(-1, 32))
    cos = (jnp.sum(sd[0, :E_NET] * sd[0, E_NET:2 * E_NET], axis=1)
           + jnp.sum(sd[1, :E_NET] * sd[1, E_NET:2 * E_NET], axis=1))
    sim = cos > SIM_THR
    trust = net_sign == 1.0
    distrust = net_sign == -1.0
    pm = trust & sim
    nm = distrust & jnp.logical_not(sim)
    om = (trust | distrust) & jnp.logical_not(pm) & jnp.logical_not(nm)
    idx_p = jnp.where(pm, net_row, DUMMY).astype(jnp.int32)
    idx_n = jnp.where(nm, net_row, DUMMY).astype(jnp.int32)
    idx_o = jnp.where(om, net_row, DUMMY).astype(jnp.int32)

    dp = (1.0 / (histo(_hidx(idx_p, EP_H_NET)) + 1e-07))[:, None]
    dn = (1.0 / (histo(_hidx(idx_n, EP_H_NET)) + 1e-07))[:, None]
    do = (1.0 / (histo(_hidx(idx_o, EP_H_NET)) + 1e-07))[:, None]

    ir_p = _sidx(net_col, idx_p, EP_NET)
    ir_n = _sidx(net_col, idx_n, EP_NET)
    ir_o = _sidx(net_col, idx_o, EP_NET)
    Sp = lambda X: segsum(ir_p, X)
    Sn = lambda X: segsum(ir_n, X)
    So = lambda X: segsum(ir_o, X)

    def graph_conv(E):
        p = dp * Sp(E)
        n = dn * Sn(E)
        o = do * So(E)
        p1 = (dp * Sp(p) + dn * Sn(n)) * 0.5
        n1 = (dn * Sn(p) + dp * Sp(n)) * 0.5
        o1 = (dp * Sp(o) + dn * Sn(o) + do * So(p + n + o)) * 0.2
        return p1, n1, o1

    P_tot = N_tot = O_tot = None
    cur = E0
    for k in range(GNN_K):
        P, N, O = graph_conv(cur)
        if P_tot is None:
            P_tot, N_tot, O_tot = P, N, O
        else:
            P_tot = P_tot + P
            N_tot = N_tot + N
            O_tot = O_tot + O
        cur = (P + N + O) / 3.0
    P = P_tot / GNN_K
    N = N_tot / GNN_K
    O = O_tot / GNN_K
    user_pn = _lin(jnp.concatenate([P, N], axis=1), Wpn, bpn)
    user_final = _lin(jnp.concatenate([user_pn, O], axis=1), Wu, bu)
    return user_final, item_init


# speedup vs baseline: 1.0780x; 1.0780x over previous
import functools

import jax
import jax.numpy as jnp
from jax import lax
from jax.experimental import pallas as pl
from jax.experimental.pallas import tpu as pltpu
from jax.experimental.pallas import tpu_sc as plsc

N_USERS = 50000
N_ITEMS = 50000
D = 64
GNN_LAYERS = 2
GNN_K = 2
SIM_THR = 0.5
E_UI = 800000
E_SOC = 800000
E_NET = E_SOC + N_USERS

NSUB = 16
BLK = 128
BLK_S = 512
CH_G = 8
CH_H = 16
CHUNK_S = NSUB * BLK_S
CHUNK_G = NSUB * CH_G * BLK * 2
CHUNK_H = NSUB * CH_H * BLK
NB = 50048
STRIPE = NB // NSUB
DUMMY = 50000


def _cdiv(a, b):
    return (a + b - 1) // b


def _mesh():
    return plsc.VectorSubcoreMesh(core_axis_name="c", subcore_axis_name="s")


_CPARAMS = pltpu.CompilerParams(use_tc_tiling_on_sc=False)


@functools.lru_cache(maxsize=None)
def _segsum_kernel(b, hx):
    tsteps = b // NSUB
    assert tsteps * NSUB == b

    def body(gidx, rows, x, zeros, out, acc, idx_v, row_v, buf, sem):
        c = lax.axis_index("c")
        s = lax.axis_index("s")
        r0 = s * STRIPE
        pltpu.sync_copy(zeros.at[pl.ds(r0, STRIPE)], acc.at[pl.ds(r0, STRIPE)])
        plsc.subcore_barrier()

        def step(t, carry):
            blk = s * tsteps + t
            pltpu.sync_copy(gidx.at[c, blk], idx_v)
            pltpu.sync_copy(rows.at[blk], row_v)
            pltpu.async_copy(x.at[idx_v], buf, sem).wait()
            pltpu.sync_copy(buf, acc.at[row_v], add=True)
            return carry

        lax.fori_loop(0, tsteps, step, 0)
        plsc.subcore_barrier()
        pltpu.sync_copy(acc.at[pl.ds(r0, STRIPE)], out.at[c, pl.ds(r0, STRIPE)])

    return pl.kernel(
        body,
        out_type=jax.ShapeDtypeStruct((2, NB, 32), jnp.float32),
        mesh=_mesh(),
        compiler_params=_CPARAMS,
        scratch_types=[
            pltpu.VMEM_SHARED((NB, 32), jnp.float32),
            pltpu.VMEM((BLK_S,), jnp.int32),
            pltpu.VMEM((BLK_S,), jnp.int32),
            pltpu.VMEM((BLK_S, 32), jnp.float32),
            pltpu.SemaphoreType.DMA,
        ],
    )


@functools.lru_cache(maxsize=None)
def _histo_kernel(bh):
    tsteps = bh // (NSUB * CH_H)
    assert tsteps * NSUB * CH_H == bh

    def body(rows, ones_h, zeros, out, acc, row_v, ones_v, ssem):
        c = lax.axis_index("c")
        s = lax.axis_index("s")
        r0 = s * STRIPE
        pltpu.sync_copy(zeros.at[pl.ds(r0, STRIPE)], acc.at[pl.ds(r0, STRIPE)])
        pltpu.sync_copy(ones_h, ones_v)
        plsc.subcore_barrier()

        def step(t, carry):
            blk0 = (s * tsteps + t) * CH_H
            pltpu.sync_copy(rows.at[c, pl.ds(blk0, CH_H)], row_v)
            cps = [
                pltpu.async_copy(ones_v, acc.at[row_v.at[j]], ssem, add=True)
                for j in range(CH_H)
            ]
            for cp in cps:
                cp.wait()
            return carry

        lax.fori_loop(0, tsteps, step, 0)
        plsc.subcore_barrier()
        pltpu.sync_copy(acc.at[pl.ds(r0, STRIPE)], out.at[c, pl.ds(r0, STRIPE)])

    return pl.kernel(
        body,
        out_type=jax.ShapeDtypeStruct((2, NB, 16), jnp.float32),
        mesh=_mesh(),
        compiler_params=_CPARAMS,
        scratch_types=[
            pltpu.VMEM_SHARED((NB, 16), jnp.float32),
            pltpu.VMEM((CH_H, BLK), jnp.int32),
            pltpu.VMEM((BLK, 16), jnp.float32),
            pltpu.SemaphoreType.DMA,
        ],
    )


@functools.lru_cache(maxsize=None)
def _gather_kernel(b128, hx):
    tsteps = b128 // (NSUB * CH_G)
    assert tsteps * NSUB * CH_G == b128 and tsteps % 2 == 0

    def body(gidx, x, out, idx0, buf0, idx1, buf1, gs0, gs1, ws0, ws1):
        c = lax.axis_index("c")
        s = lax.axis_index("s")
        base = s * tsteps * CH_G

        def stage(blk, idxv):
            pltpu.sync_copy(gidx.at[c, pl.ds(blk, CH_G)], idxv)

        def fire_g(idxv, bufv, sem):
            for j in range(CH_G):
                pltpu.async_copy(x.at[idxv.at[j]], bufv.at[pl.ds(j * BLK, BLK)], sem)

        def wait_g(idxv, bufv, sem):
            for j in range(CH_G):
                pltpu.make_async_copy(x.at[idxv.at[j]], bufv.at[pl.ds(j * BLK, BLK)], sem).wait()

        def fire_w(blk, bufv, sem):
            pltpu.async_copy(bufv, out.at[c, pl.ds(blk * BLK, CH_G * BLK)], sem)

        def wait_w(blk, bufv, sem):
            pltpu.make_async_copy(bufv, out.at[c, pl.ds(blk * BLK, CH_G * BLK)], sem).wait()

        stage(base, idx0)
        fire_g(idx0, buf0, gs0)

        def it(tt, carry):
            blk = base + 2 * tt * CH_G

            @pl.when(tt >= 1)
            def _():
                wait_w(blk - CH_G, buf1, ws1)

            stage(blk + CH_G, idx1)
            fire_g(idx1, buf1, gs1)
            wait_g(idx0, buf0, gs0)
            fire_w(blk, buf0, ws0)

            @pl.when(tt < tsteps // 2 - 1)
            def _():
                wait_w(blk, buf0, ws0)
                stage(blk + 2 * CH_G, idx0)
                fire_g(idx0, buf0, gs0)

            wait_g(idx1, buf1, gs1)
            fire_w(blk + CH_G, buf1, ws1)
            return carry

        lax.fori_loop(0, tsteps // 2, it, 0)
        wait_w(base + (tsteps - 2) * CH_G, buf0, ws0)
        wait_w(base + (tsteps - 1) * CH_G, buf1, ws1)

    return pl.kernel(
        body,
        out_type=jax.ShapeDtypeStruct((2, b128 * BLK, 32), jnp.float32),
        mesh=_mesh(),
        compiler_params=_CPARAMS,
        scratch_types=[
            pltpu.VMEM((CH_G, BLK), jnp.int32),
            pltpu.VMEM((CH_G * BLK, 32), jnp.float32),
            pltpu.VMEM((CH_G, BLK), jnp.int32),
            pltpu.VMEM((CH_G * BLK, 32), jnp.float32),
            pltpu.SemaphoreType.DMA,
            pltpu.SemaphoreType.DMA,
            pltpu.SemaphoreType.DMA,
            pltpu.SemaphoreType.DMA,
        ],
    )


def _pad1(x, n, val):
    e = x.shape[0]
    if n == e:
        return x
    return jnp.concatenate([x, jnp.full((n - e,), val, x.dtype)])


def _gidx(col, n, blk=BLK):
    g = _pad1(col, n, 0).astype(jnp.int32) * 2
    return jnp.stack([g, g + 1]).reshape(2, n // blk, blk)


def _ridx(row, n, blk=BLK):
    return _pad1(row, n, DUMMY).astype(jnp.int32).reshape(n // blk, blk)


def _hidx(row, nh):
    e = row.shape[0]
    h0 = _pad1(row[: e // 2], nh, DUMMY).astype(jnp.int32)
    h1 = _pad1(row[e // 2:], nh, DUMMY).astype(jnp.int32)
    return jnp.stack([h0, h1]).reshape(2, nh // BLK, BLK)


def _lin(x, W, b):
    return x @ W.T + b


EP_UI = _cdiv(E_UI, CHUNK_S) * CHUNK_S
EP_NET = _cdiv(E_NET, CHUNK_S) * CHUNK_S
EP_G = _cdiv(2 * E_NET, CHUNK_G) * CHUNK_G
EP_H_UI = _cdiv(E_UI // 2, CHUNK_H) * CHUNK_H
EP_H_NET = _cdiv((E_NET + 1) // 2, CHUNK_H) * CHUNK_H


def kernel(user_emb, item_emb, Wuc, buc, Wic, bic, Wui, bui, Wpn, bpn, Wu, bu,
           ui_row, ui_col, soc_row, soc_col, soc_sign):
    f32 = jnp.float32
    diag = jnp.arange(N_USERS, dtype=soc_row.dtype)
    net_row = jnp.concatenate([soc_row, diag])
    net_col = jnp.concatenate([soc_col, diag])
    net_sign = jnp.concatenate([soc_sign.astype(f32), jnp.ones((N_USERS,), f32)])

    zeros32 = jnp.zeros((NB, 32), f32)
    zeros16 = jnp.zeros((NB, 16), f32)
    ones16 = jnp.ones((BLK, 16), f32)

    def histo(rows3):
        out = _histo_kernel(rows3.shape[1])(rows3, ones16, zeros16)
        return out[0, :N_USERS, 0] + out[1, :N_USERS, 0]

    def segsum(gidx3, rows3, x64):
        x2 = x64.reshape(-1, 32)
        out = _segsum_kernel(gidx3.shape[1], x2.shape[0])(gidx3, rows3, x2, zeros32)
        return jnp.concatenate([out[0, :N_USERS], out[1, :N_USERS]], axis=1)

    g_ui_col = _gidx(ui_col, EP_UI, BLK_S)
    g_ui_row = _gidx(ui_row, EP_UI, BLK_S)
    r_ui_row = _ridx(ui_row, EP_UI, BLK_S)
    r_ui_col = _ridx(ui_col, EP_UI, BLK_S)
    dinv_u = (1.0 / (histo(_hidx(ui_row, EP_H_UI)) + 1e-07))[:, None]
    dinv_i = (1.0 / (histo(_hidx(ui_col, EP_H_UI)) + 1e-07))[:, None]

    ACM = lambda X: dinv_u * segsum(g_ui_col, r_ui_row, X)
    ACM_T = lambda X: dinv_i * segsum(g_ui_row, r_ui_col, X)

    item_rep0 = ACM(item_emb)
    user_rep0 = ACM_T(user_emb)
    item_init = _lin(jnp.concatenate([item_emb, ACM_T(item_rep0)], axis=1), Wic, bic)
    user_init = _lin(jnp.concatenate([user_emb, ACM(user_rep0)], axis=1), Wuc, buc)
    item_rep = ACM(item_init)
    E0 = _lin(jnp.concatenate([user_init, item_rep], axis=1), Wui, bui)

    invn = 1.0 / jnp.maximum(jnp.sqrt(jnp.sum(item_rep * item_rep, axis=1)), 1e-08)
    irn = item_rep * invn[:, None]
    gcat = _gidx(jnp.concatenate([net_row, net_col]), EP_G)
    sd = _gather_kernel(EP_G // BLK, 2 * N_ITEMS)(gcat, irn.reshape(-1, 32))
    cos = (jnp.sum(sd[0, :E_NET] * sd[0, E_NET:2 * E_NET], axis=1)
           + jnp.sum(sd[1, :E_NET] * sd[1, E_NET:2 * E_NET], axis=1))
    sim = cos > SIM_THR
    trust = net_sign == 1.0
    distrust = net_sign == -1.0
    pm = trust & sim
    nm = distrust & jnp.logical_not(sim)
    om = (trust | distrust) & jnp.logical_not(pm) & jnp.logical_not(nm)
    idx_p = jnp.where(pm, net_row, DUMMY).astype(jnp.int32)
    idx_n = jnp.where(nm, net_row, DUMMY).astype(jnp.int32)
    idx_o = jnp.where(om, net_row, DUMMY).astype(jnp.int32)

    dp = (1.0 / (histo(_hidx(idx_p, EP_H_NET)) + 1e-07))[:, None]
    dn = (1.0 / (histo(_hidx(idx_n, EP_H_NET)) + 1e-07))[:, None]
    do = (1.0 / (histo(_hidx(idx_o, EP_H_NET)) + 1e-07))[:, None]

    g_net = _gidx(net_col, EP_NET, BLK_S)
    r_p = _ridx(idx_p, EP_NET, BLK_S)
    r_n = _ridx(idx_n, EP_NET, BLK_S)
    r_o = _ridx(idx_o, EP_NET, BLK_S)
    Sp = lambda X: segsum(g_net, r_p, X)
    Sn = lambda X: segsum(g_net, r_n, X)
    So = lambda X: segsum(g_net, r_o, X)

    def graph_conv(E):
        p = dp * Sp(E)
        n = dn * Sn(E)
        o = do * So(E)
        p1 = (dp * Sp(p) + dn * Sn(n)) * 0.5
        n1 = (dn * Sn(p) + dp * Sp(n)) * 0.5
        o1 = (dp * Sp(o) + dn * Sn(o) + do * So(p + n + o)) * 0.2
        return p1, n1, o1

    P_tot = N_tot = O_tot = None
    cur = E0
    for k in range(GNN_K):
        P, N, O = graph_conv(cur)
        if P_tot is None:
            P_tot, N_tot, O_tot = P, N, O
        else:
            P_tot = P_tot + P
            N_tot = N_tot + N
            O_tot = O_tot + O
        cur = (P + N + O) / 3.0
    P = P_tot / GNN_K
    N = N_tot / GNN_K
    O = O_tot / GNN_K
    user_pn = _lin(jnp.concatenate([P, N], axis=1), Wpn, bpn)
    user_final = _lin(jnp.concatenate([user_pn, O], axis=1), Wu, bu)
    return user_final, item_init
